# Initial kernel scaffold; baseline (speedup 1.0000x reference)
import jax
import jax.numpy as jnp
from jax.experimental import pallas as pl


def kernel(x, edge_index, W1, b1, W2, b2):
    raise NotImplementedError("write your pallas kernel here")


# speedup vs baseline: 33.1063x; 33.1063x over previous
import functools

import jax
import jax.numpy as jnp
from jax import lax
from jax.experimental import pallas as pl
from jax.experimental.pallas import tpu as pltpu
from jax.experimental.pallas import tpu_sc as plsc

N = 10000
D_IN = 128
D_HID = 16
D_OUT = 64
E = 320000

NC, NS = 2, 16
NW = NC * NS
GRP = 128
G = 79
EPAD = NW * G * GRP
RPT = 632
NPAD = NS * RPT

_MESH = plsc.VectorSubcoreMesh(core_axis_name="c", subcore_axis_name="s")
_f32 = jnp.float32


def _hist_body(dst_hbm, ones_hbm, z_hbm, out_hbm, dst_v, ones_v, acc_sp):
    cid = lax.axis_index("c")
    sid = lax.axis_index("s")
    wid = sid * NC + cid
    pltpu.sync_copy(dst_hbm.at[wid], dst_v)
    pltpu.sync_copy(ones_hbm, ones_v)
    r0 = sid * RPT
    pltpu.sync_copy(z_hbm.at[pl.ds(r0, RPT)], acc_sp.at[pl.ds(r0, RPT)])
    plsc.subcore_barrier()

    @pl.loop(0, G)
    def _(g):
        pltpu.sync_copy(ones_v, acc_sp.at[dst_v.at[g]], add=True)

    plsc.subcore_barrier()
    pltpu.sync_copy(acc_sp.at[pl.ds(r0, RPT)], out_hbm.at[cid, pl.ds(r0, RPT)])


_hist_call = pl.kernel(
    _hist_body,
    out_type=jax.ShapeDtypeStruct((NC, NPAD, D_HID), _f32),
    mesh=_MESH,
    compiler_params=pltpu.CompilerParams(use_tc_tiling_on_sc=False),
    scratch_types=[
        pltpu.VMEM((G, GRP), jnp.int32),
        pltpu.VMEM((GRP, D_HID), _f32),
        pltpu.VMEM_SHARED((NPAD, D_HID), _f32),
    ],
)


def _scat_body(tab_hbm, src_hbm, dst_hbm, z_hbm, out_hbm,
               src_v, dst_v, rows_v, acc_sp, sem):
    cid = lax.axis_index("c")
    sid = lax.axis_index("s")
    wid = sid * NC + cid
    pltpu.sync_copy(src_hbm.at[wid], src_v)
    pltpu.sync_copy(dst_hbm.at[wid], dst_v)
    r0 = sid * RPT
    pltpu.sync_copy(z_hbm.at[pl.ds(r0, RPT)], acc_sp.at[pl.ds(r0, RPT)])
    plsc.subcore_barrier()

    @pl.loop(0, G)
    def _(g):
        pltpu.async_copy(tab_hbm.at[src_v.at[g]], rows_v, sem).wait()
        pltpu.sync_copy(rows_v, acc_sp.at[dst_v.at[g]], add=True)

    plsc.subcore_barrier()
    pltpu.sync_copy(acc_sp.at[pl.ds(r0, RPT)], out_hbm.at[cid, pl.ds(r0, RPT)])


_scat_call = pl.kernel(
    _scat_body,
    out_type=jax.ShapeDtypeStruct((NC, NPAD, D_HID), _f32),
    mesh=_MESH,
    compiler_params=pltpu.CompilerParams(use_tc_tiling_on_sc=False),
    scratch_types=[
        pltpu.VMEM((G, GRP), jnp.int32),
        pltpu.VMEM((G, GRP), jnp.int32),
        pltpu.VMEM((GRP, D_HID), _f32),
        pltpu.VMEM_SHARED((NPAD, D_HID), _f32),
        pltpu.SemaphoreType.DMA,
    ],
)


def _mm1_body(x_ref, w_ref, o_ref):
    o_ref[...] = jnp.dot(x_ref[...], w_ref[...], preferred_element_type=_f32)


def _s1_body(h1_ref, p_ref, o_h1p, o_dinv):
    dinv = lax.rsqrt(1.0 + p_ref[0] + p_ref[1])
    o_dinv[...] = dinv
    o_h1p[...] = h1_ref[...] * dinv


def _l1_body(t_ref, h1p_ref, dinv_ref, b1_ref, o_ref):
    s = t_ref[0] + t_ref[1] + h1p_ref[...]
    a = jnp.maximum(b1_ref[...] + dinv_ref[...] * s, 0.0)
    o_ref[...] = a * dinv_ref[...]


def _l2_body(t_ref, a1p_ref, dinv_ref, w2_ref, b2_ref, o_ref):
    u = t_ref[0] + t_ref[1] + a1p_ref[...]
    v = jnp.dot(u, w2_ref[...], preferred_element_type=_f32)
    dinv64 = jnp.broadcast_to(dinv_ref[:, 0:1], (NPAD, D_OUT))
    o = b2_ref[...] + dinv64 * v
    m = jnp.max(o, axis=1, keepdims=True)
    lse = jnp.log(jnp.sum(jnp.exp(o - m), axis=1, keepdims=True)) + m
    o_ref[...] = o - lse


_mm1_call = pl.pallas_call(
    _mm1_body, out_shape=jax.ShapeDtypeStruct((NPAD, D_HID), _f32))
_s1_call = pl.pallas_call(
    _s1_body, out_shape=(jax.ShapeDtypeStruct((NPAD, D_HID), _f32),
                         jax.ShapeDtypeStruct((NPAD, D_HID), _f32)))
_l1_call = pl.pallas_call(
    _l1_body, out_shape=jax.ShapeDtypeStruct((NPAD, D_HID), _f32))
_l2_call = pl.pallas_call(
    _l2_body, out_shape=jax.ShapeDtypeStruct((NPAD, D_OUT), _f32))


def kernel(x, edge_index, W1, b1, W2, b2):
    src = edge_index[0]
    dst = edge_index[1]
    pad = EPAD - E
    srcp = jnp.concatenate([src, jnp.zeros((pad,), src.dtype)]).reshape(NW, G, GRP)
    dstp = jnp.concatenate([dst, jnp.full((pad,), N, dst.dtype)]).reshape(NW, G, GRP)
    xp = jnp.pad(x, ((0, NPAD - N), (0, 0)))
    z16 = jnp.zeros((NPAD, D_HID), _f32)
    ones_rows = jnp.ones((GRP, D_HID), _f32)

    hist = _hist_call(dstp, ones_rows, z16)
    h1 = _mm1_call(xp, W1)
    h1p, dinv = _s1_call(h1, hist)
    t1 = _scat_call(h1p, srcp, dstp, z16)
    a1p = _l1_call(t1, h1p, dinv, b1.reshape(1, D_HID))
    t2 = _scat_call(a1p, srcp, dstp, z16)
    out = _l2_call(t2, a1p, dinv, W2, b2.reshape(1, D_OUT))
    return out[:N]


# speedup vs baseline: 38.7848x; 1.1715x over previous
import functools

import jax
import jax.numpy as jnp
from jax import lax
from jax.experimental import pallas as pl
from jax.experimental.pallas import tpu as pltpu
from jax.experimental.pallas import tpu_sc as plsc

N = 10000
D_IN = 128
D_HID = 16
D_OUT = 64
E = 320000

NC, NS = 2, 16
NW = NC * NS
GRP = 128
KB = 4
NCH = 20
G = NCH * KB
EPAD = NW * G * GRP
RPT = 632
NPAD = NS * RPT

_MESH = plsc.VectorSubcoreMesh(core_axis_name="c", subcore_axis_name="s")
_f32 = jnp.float32


def _hist_body(dst_hbm, ones_hbm, z_hbm, out_hbm, dst_v, ones_v, acc_sp, sem):
    cid = lax.axis_index("c")
    sid = lax.axis_index("s")
    wid = sid * NC + cid
    pltpu.sync_copy(dst_hbm.at[wid], dst_v)
    pltpu.sync_copy(ones_hbm, ones_v)
    r0 = sid * RPT
    pltpu.sync_copy(z_hbm.at[pl.ds(r0, RPT)], acc_sp.at[pl.ds(r0, RPT)])
    plsc.subcore_barrier()

    @pl.loop(0, G)
    def _(g):
        pltpu.async_copy(ones_v, acc_sp.at[dst_v.at[g]], sem, add=True)

    @pl.loop(0, G)
    def _(g):
        pltpu.make_async_copy(ones_hbm, ones_v, sem).wait()

    plsc.subcore_barrier()
    pltpu.sync_copy(acc_sp.at[pl.ds(r0, RPT)], out_hbm.at[cid, pl.ds(r0, RPT)])


_hist_call = pl.kernel(
    _hist_body,
    out_type=jax.ShapeDtypeStruct((NC, NPAD, D_HID), _f32),
    mesh=_MESH,
    compiler_params=pltpu.CompilerParams(use_tc_tiling_on_sc=False),
    scratch_types=[
        pltpu.VMEM((G, GRP), jnp.int32),
        pltpu.VMEM((GRP, D_HID), _f32),
        pltpu.VMEM_SHARED((NPAD, D_HID), _f32),
        pltpu.SemaphoreType.DMA,
    ],
)


def _scat_body(tab_hbm, src_hbm, dst_hbm, z_hbm, out_hbm,
               src_v, dst_v, rows_v, acc_sp, gsems, ssems):
    cid = lax.axis_index("c")
    sid = lax.axis_index("s")
    wid = sid * NC + cid
    pltpu.sync_copy(src_hbm.at[wid], src_v)
    pltpu.sync_copy(dst_hbm.at[wid], dst_v)
    r0 = sid * RPT
    pltpu.sync_copy(z_hbm.at[pl.ds(r0, RPT)], acc_sp.at[pl.ds(r0, RPT)])
    plsc.subcore_barrier()

    def fire_gather(c, p, k):
        pltpu.async_copy(tab_hbm.at[src_v.at[c * KB + k]],
                         rows_v.at[p, k], gsems.at[p, k])

    def drain(sem_ref):
        pltpu.make_async_copy(tab_hbm.at[pl.ds(0, GRP)],
                              rows_v.at[0, 0], sem_ref).wait()

    for k in range(KB):
        fire_gather(0, 0, k)

    @pl.loop(0, NCH // 2)
    def _(i):
        for p in range(2):
            c = 2 * i + p
            for k in range(KB):
                drain(gsems.at[p, k])
            @pl.when(c >= 1)
            def _():
                for k in range(KB):
                    drain(ssems.at[1 - p, k])

            @pl.when(c + 1 < NCH)
            def _():
                for k in range(KB):
                    fire_gather(c + 1, 1 - p, k)
            for k in range(KB):
                pltpu.async_copy(rows_v.at[p, k],
                                 acc_sp.at[dst_v.at[c * KB + k]],
                                 ssems.at[p, k], add=True)

    for k in range(KB):
        drain(ssems.at[1, k])

    plsc.subcore_barrier()
    pltpu.sync_copy(acc_sp.at[pl.ds(r0, RPT)], out_hbm.at[cid, pl.ds(r0, RPT)])


_scat_call = pl.kernel(
    _scat_body,
    out_type=jax.ShapeDtypeStruct((NC, NPAD, D_HID), _f32),
    mesh=_MESH,
    compiler_params=pltpu.CompilerParams(use_tc_tiling_on_sc=False),
    scratch_types=[
        pltpu.VMEM((G, GRP), jnp.int32),
        pltpu.VMEM((G, GRP), jnp.int32),
        pltpu.VMEM((2, KB, GRP, D_HID), _f32),
        pltpu.VMEM_SHARED((NPAD, D_HID), _f32),
        pltpu.SemaphoreType.DMA((2, KB)),
        pltpu.SemaphoreType.DMA((2, KB)),
    ],
)


def _mm1_body(x_ref, w_ref, o_ref):
    o_ref[...] = jnp.dot(x_ref[...], w_ref[...], preferred_element_type=_f32)


def _s1_body(h1_ref, p_ref, o_h1p, o_dinv):
    dinv = lax.rsqrt(1.0 + p_ref[0] + p_ref[1])
    o_dinv[...] = dinv
    o_h1p[...] = h1_ref[...] * dinv


def _l1_body(t_ref, h1p_ref, dinv_ref, b1_ref, o_ref):
    s = t_ref[0] + t_ref[1] + h1p_ref[...]
    a = jnp.maximum(b1_ref[...] + dinv_ref[...] * s, 0.0)
    o_ref[...] = a * dinv_ref[...]


def _l2_body(t_ref, a1p_ref, dinv_ref, w2_ref, b2_ref, o_ref):
    u = t_ref[0] + t_ref[1] + a1p_ref[...]
    v = jnp.dot(u, w2_ref[...], preferred_element_type=_f32)
    dinv64 = jnp.broadcast_to(dinv_ref[:, 0:1], (NPAD, D_OUT))
    o = b2_ref[...] + dinv64 * v
    m = jnp.max(o, axis=1, keepdims=True)
    lse = jnp.log(jnp.sum(jnp.exp(o - m), axis=1, keepdims=True)) + m
    o_ref[...] = o - lse


_mm1_call = pl.pallas_call(
    _mm1_body, out_shape=jax.ShapeDtypeStruct((NPAD, D_HID), _f32))
_s1_call = pl.pallas_call(
    _s1_body, out_shape=(jax.ShapeDtypeStruct((NPAD, D_HID), _f32),
                         jax.ShapeDtypeStruct((NPAD, D_HID), _f32)))
_l1_call = pl.pallas_call(
    _l1_body, out_shape=jax.ShapeDtypeStruct((NPAD, D_HID), _f32))
_l2_call = pl.pallas_call(
    _l2_body, out_shape=jax.ShapeDtypeStruct((NPAD, D_OUT), _f32))


def kernel(x, edge_index, W1, b1, W2, b2):
    src = edge_index[0]
    dst = edge_index[1]
    pad = EPAD - E
    srcp = jnp.concatenate([src, jnp.zeros((pad,), src.dtype)]).reshape(NW, G, GRP)
    dstp = jnp.concatenate([dst, jnp.full((pad,), N, dst.dtype)]).reshape(NW, G, GRP)
    xp = jnp.pad(x, ((0, NPAD - N), (0, 0)))
    z16 = jnp.zeros((NPAD, D_HID), _f32)
    ones_rows = jnp.ones((GRP, D_HID), _f32)

    hist = _hist_call(dstp, ones_rows, z16)
    h1 = _mm1_call(xp, W1)
    h1p, dinv = _s1_call(h1, hist)
    t1 = _scat_call(h1p, srcp, dstp, z16)
    a1p = _l1_call(t1, h1p, dinv, b1.reshape(1, D_HID))
    t2 = _scat_call(a1p, srcp, dstp, z16)
    out = _l2_call(t2, a1p, dinv, W2, b2.reshape(1, D_OUT))
    return out[:N]


# speedup vs baseline: 52.7051x; 1.3589x over previous
import functools

import jax
import jax.numpy as jnp
from jax import lax
from jax.experimental import pallas as pl
from jax.experimental.pallas import tpu as pltpu
from jax.experimental.pallas import tpu_sc as plsc

N = 10000
D_IN = 128
D_HID = 16
D_OUT = 64
E = 320000

NC, NS = 2, 16
NW = NC * NS
GRP = 128
KB = 4
NCH = 20
G = NCH * KB
EPAD = NW * G * GRP
RPT = 632
NPAD = NS * RPT

_MESH = plsc.VectorSubcoreMesh(core_axis_name="c", subcore_axis_name="s")
_f32 = jnp.float32


def _hist_body(dst_hbm, ones_hbm, z_hbm, out_hbm, dst_v, ones_v, acc_sp, sem):
    cid = lax.axis_index("c")
    sid = lax.axis_index("s")
    wid = sid * NC + cid
    pltpu.sync_copy(dst_hbm.at[wid], dst_v)
    pltpu.sync_copy(ones_hbm, ones_v)
    r0 = sid * RPT
    pltpu.sync_copy(z_hbm.at[pl.ds(r0, RPT)], acc_sp.at[pl.ds(r0, RPT)])
    plsc.subcore_barrier()

    @pl.loop(0, G)
    def _(g):
        pltpu.async_copy(ones_v, acc_sp.at[dst_v.at[g]], sem, add=True)

    @pl.loop(0, G)
    def _(g):
        pltpu.make_async_copy(ones_hbm, ones_v, sem).wait()

    plsc.subcore_barrier()
    pltpu.sync_copy(acc_sp.at[pl.ds(r0, RPT)], out_hbm.at[cid, pl.ds(r0, RPT)])


_hist_call = pl.kernel(
    _hist_body,
    out_type=jax.ShapeDtypeStruct((NC, NPAD, D_HID), _f32),
    mesh=_MESH,
    compiler_params=pltpu.CompilerParams(use_tc_tiling_on_sc=False),
    scratch_types=[
        pltpu.VMEM((G, GRP), jnp.int32),
        pltpu.VMEM((GRP, D_HID), _f32),
        pltpu.VMEM_SHARED((NPAD, D_HID), _f32),
        pltpu.SemaphoreType.DMA,
    ],
)


def _scat_body(tab_hbm, src_hbm, dst_hbm, z_hbm, out_hbm,
               src_v, dst_v, rows_v, tab_sp, acc_sp, gsems, ssems):
    cid = lax.axis_index("c")
    sid = lax.axis_index("s")
    wid = sid * NC + cid
    pltpu.sync_copy(src_hbm.at[wid], src_v)
    pltpu.sync_copy(dst_hbm.at[wid], dst_v)
    r0 = sid * RPT
    pltpu.sync_copy(z_hbm.at[pl.ds(r0, RPT)], acc_sp.at[pl.ds(r0, RPT)])
    pltpu.sync_copy(tab_hbm.at[pl.ds(r0, RPT)], tab_sp.at[pl.ds(r0, RPT)])
    plsc.subcore_barrier()

    def fire_gather(c, p, k):
        pltpu.async_copy(tab_sp.at[src_v.at[c * KB + k]],
                         rows_v.at[p, k], gsems.at[p, k])

    def drain(sem_ref):
        pltpu.make_async_copy(tab_hbm.at[pl.ds(0, GRP)],
                              rows_v.at[0, 0], sem_ref).wait()

    for k in range(KB):
        fire_gather(0, 0, k)

    @pl.loop(0, NCH // 2)
    def _(i):
        for p in range(2):
            c = 2 * i + p
            for k in range(KB):
                drain(gsems.at[p, k])
            @pl.when(c >= 1)
            def _():
                for k in range(KB):
                    drain(ssems.at[1 - p, k])

            @pl.when(c + 1 < NCH)
            def _():
                for k in range(KB):
                    fire_gather(c + 1, 1 - p, k)
            for k in range(KB):
                pltpu.async_copy(rows_v.at[p, k],
                                 acc_sp.at[dst_v.at[c * KB + k]],
                                 ssems.at[p, k], add=True)

    for k in range(KB):
        drain(ssems.at[1, k])

    plsc.subcore_barrier()
    pltpu.sync_copy(acc_sp.at[pl.ds(r0, RPT)], out_hbm.at[cid, pl.ds(r0, RPT)])


_scat_call = pl.kernel(
    _scat_body,
    out_type=jax.ShapeDtypeStruct((NC, NPAD, D_HID), _f32),
    mesh=_MESH,
    compiler_params=pltpu.CompilerParams(use_tc_tiling_on_sc=False),
    scratch_types=[
        pltpu.VMEM((G, GRP), jnp.int32),
        pltpu.VMEM((G, GRP), jnp.int32),
        pltpu.VMEM((2, KB, GRP, D_HID), _f32),
        pltpu.VMEM_SHARED((NPAD, D_HID), _f32),
        pltpu.VMEM_SHARED((NPAD, D_HID), _f32),
        pltpu.SemaphoreType.DMA((2, KB)),
        pltpu.SemaphoreType.DMA((2, KB)),
    ],
)


def _pre_body(x_ref, w_ref, p_ref, o_h1p, o_dinv):
    dinv = lax.rsqrt(1.0 + p_ref[0] + p_ref[1])
    o_dinv[...] = dinv
    h1 = jnp.dot(x_ref[...], w_ref[...], preferred_element_type=_f32)
    o_h1p[pl.ds(0, N), :] = h1 * dinv[:N]
    o_h1p[pl.ds(N, NPAD - N), :] = jnp.zeros((NPAD - N, D_HID), _f32)


def _l1_body(t_ref, h1p_ref, dinv_ref, b1_ref, o_ref):
    s = t_ref[0] + t_ref[1] + h1p_ref[...]
    a = jnp.maximum(b1_ref[...] + dinv_ref[...] * s, 0.0)
    o_ref[...] = a * dinv_ref[...]


def _l2_body(t_ref, a1p_ref, dinv_ref, w2_ref, b2_ref, o_ref):
    u = (t_ref[0] + t_ref[1] + a1p_ref[...])[:N]
    v = jnp.dot(u, w2_ref[...], preferred_element_type=_f32)
    dinv64 = jnp.broadcast_to(dinv_ref[:N, 0:1], (N, D_OUT))
    o = b2_ref[...] + dinv64 * v
    m = jnp.max(o, axis=1, keepdims=True)
    lse = jnp.log(jnp.sum(jnp.exp(o - m), axis=1, keepdims=True)) + m
    o_ref[...] = o - lse


_pre_call = pl.pallas_call(
    _pre_body, out_shape=(jax.ShapeDtypeStruct((NPAD, D_HID), _f32),
                          jax.ShapeDtypeStruct((NPAD, D_HID), _f32)))
_l1_call = pl.pallas_call(
    _l1_body, out_shape=jax.ShapeDtypeStruct((NPAD, D_HID), _f32))
_l2_call = pl.pallas_call(
    _l2_body, out_shape=jax.ShapeDtypeStruct((N, D_OUT), _f32))


def kernel(x, edge_index, W1, b1, W2, b2):
    src = edge_index[0]
    dst = edge_index[1]
    pad = EPAD - E
    srcp = jnp.concatenate([src, jnp.zeros((pad,), src.dtype)]).reshape(NW, G, GRP)
    dstp = jnp.concatenate([dst, jnp.full((pad,), N, dst.dtype)]).reshape(NW, G, GRP)
    z16 = jnp.zeros((NPAD, D_HID), _f32)
    ones_rows = jnp.ones((GRP, D_HID), _f32)

    hist = _hist_call(dstp, ones_rows, z16)
    h1p, dinv = _pre_call(x, W1, hist)
    t1 = _scat_call(h1p, srcp, dstp, z16)
    a1p = _l1_call(t1, h1p, dinv, b1.reshape(1, D_HID))
    t2 = _scat_call(a1p, srcp, dstp, z16)
    out = _l2_call(t2, a1p, dinv, W2, b2.reshape(1, D_OUT))
    return out


# speedup vs baseline: 61.3376x; 1.1638x over previous
import functools

import jax
import jax.numpy as jnp
from jax import lax
from jax.experimental import pallas as pl
from jax.experimental.pallas import tpu as pltpu
from jax.experimental.pallas import tpu_sc as plsc

N = 10000
D_IN = 128
D_HID = 16
D_OUT = 64
E = 320000

NC, NS = 2, 16
NW = NC * NS
GRP = 128
KB = 4
NCH = 20
G = NCH * KB
NGE = E // GRP
NWX = 17
RPT = 632
NPAD = NS * RPT

_MESH = plsc.VectorSubcoreMesh(core_axis_name="c", subcore_axis_name="s")
_f32 = jnp.float32


def _load_idx(ei_hbm, plane, wid, idx_v, pad_val):
    gb = 4 * (19 * wid + jnp.minimum(wid, NWX))
    fill = jnp.full((16,), pad_val, jnp.int32)
    for r in range(G - 4, G):
        for c in range(GRP // 16):
            idx_v[r, pl.ds(c * 16, 16)] = fill
    pltpu.sync_copy(ei_hbm.at[plane, pl.ds(gb, G - 4)], idx_v.at[pl.ds(0, G - 4)])

    @pl.when(wid < NWX)
    def _():
        pltpu.sync_copy(ei_hbm.at[plane, pl.ds(gb + G - 4, 4)],
                        idx_v.at[pl.ds(G - 4, 4)])


def _hist_body(ei_hbm, ones_hbm, z_hbm, out_hbm, dst_v, ones_v, acc_sp, sem):
    cid = lax.axis_index("c")
    sid = lax.axis_index("s")
    wid = sid * NC + cid
    _load_idx(ei_hbm, 1, wid, dst_v, N)
    pltpu.sync_copy(ones_hbm, ones_v)
    r0 = sid * RPT
    pltpu.sync_copy(z_hbm.at[pl.ds(r0, RPT)], acc_sp.at[pl.ds(r0, RPT)])
    plsc.subcore_barrier()

    @pl.loop(0, G)
    def _(g):
        pltpu.async_copy(ones_v, acc_sp.at[dst_v.at[g]], sem, add=True)

    @pl.loop(0, G)
    def _(g):
        pltpu.make_async_copy(ones_hbm, ones_v, sem).wait()

    plsc.subcore_barrier()
    pltpu.sync_copy(acc_sp.at[pl.ds(r0, RPT)], out_hbm.at[cid, pl.ds(r0, RPT)])


_hist_call = pl.kernel(
    _hist_body,
    out_type=jax.ShapeDtypeStruct((NC, NPAD, D_HID), _f32),
    mesh=_MESH,
    compiler_params=pltpu.CompilerParams(use_tc_tiling_on_sc=False),
    scratch_types=[
        pltpu.VMEM((G, GRP), jnp.int32),
        pltpu.VMEM((GRP, D_HID), _f32),
        pltpu.VMEM_SHARED((NPAD, D_HID), _f32),
        pltpu.SemaphoreType.DMA,
    ],
)


def _edge_pipeline(tab_hbm, src_v, dst_v, rows_v, tab_sp, acc_sp, gsems, ssems):
    def fire_gather(c, p, k):
        pltpu.async_copy(tab_sp.at[src_v.at[c * KB + k]],
                         rows_v.at[p, k], gsems.at[p, k])

    def drain(sem_ref):
        pltpu.make_async_copy(tab_hbm.at[pl.ds(0, GRP)],
                              rows_v.at[0, 0], sem_ref).wait()

    for k in range(KB):
        fire_gather(0, 0, k)

    @pl.loop(0, NCH // 2)
    def _(i):
        for p in range(2):
            c = 2 * i + p
            for k in range(KB):
                drain(gsems.at[p, k])
            @pl.when(c >= 1)
            def _():
                for k in range(KB):
                    drain(ssems.at[1 - p, k])

            @pl.when(c + 1 < NCH)
            def _():
                for k in range(KB):
                    fire_gather(c + 1, 1 - p, k)
            for k in range(KB):
                pltpu.async_copy(rows_v.at[p, k],
                                 acc_sp.at[dst_v.at[c * KB + k]],
                                 ssems.at[p, k], add=True)

    for k in range(KB):
        drain(ssems.at[1, k])


_SCAT_SCRATCH = [
    pltpu.VMEM((G, GRP), jnp.int32),
    pltpu.VMEM((G, GRP), jnp.int32),
    pltpu.VMEM((2, KB, GRP, D_HID), _f32),
    pltpu.VMEM_SHARED((NPAD, D_HID), _f32),
    pltpu.VMEM_SHARED((NPAD, D_HID), _f32),
    pltpu.SemaphoreType.DMA((2, KB)),
    pltpu.SemaphoreType.DMA((2, KB)),
]


def _scat1_body(tab_hbm, ei_hbm, z_hbm, out_hbm,
                src_v, dst_v, rows_v, tab_sp, acc_sp, gsems, ssems):
    cid = lax.axis_index("c")
    sid = lax.axis_index("s")
    wid = sid * NC + cid
    _load_idx(ei_hbm, 0, wid, src_v, 0)
    _load_idx(ei_hbm, 1, wid, dst_v, N)
    r0 = sid * RPT
    pltpu.sync_copy(z_hbm.at[pl.ds(r0, RPT)], acc_sp.at[pl.ds(r0, RPT)])
    pltpu.sync_copy(tab_hbm.at[pl.ds(r0, RPT)], tab_sp.at[pl.ds(r0, RPT)])
    plsc.subcore_barrier()
    _edge_pipeline(tab_hbm, src_v, dst_v, rows_v, tab_sp, acc_sp, gsems, ssems)
    plsc.subcore_barrier()
    pltpu.sync_copy(acc_sp.at[pl.ds(r0, RPT)], out_hbm.at[cid, pl.ds(r0, RPT)])


_scat1_call = pl.kernel(
    _scat1_body,
    out_type=jax.ShapeDtypeStruct((NC, NPAD, D_HID), _f32),
    mesh=_MESH,
    compiler_params=pltpu.CompilerParams(use_tc_tiling_on_sc=False),
    scratch_types=_SCAT_SCRATCH,
)


def _scat2_body(t1_hbm, h1p_hbm, dinv_hbm, b1_hbm, ei_hbm, z_hbm,
                out_hbm, tab_out_hbm,
                src_v, dst_v, rows_v, tab_sp, acc_sp, gsems, ssems,
                p0_v, p1_v, h1p_v, dinv_v, b1_v, tr_v):
    cid = lax.axis_index("c")
    sid = lax.axis_index("s")
    wid = sid * NC + cid
    _load_idx(ei_hbm, 0, wid, src_v, 0)
    _load_idx(ei_hbm, 1, wid, dst_v, N)
    r0 = sid * RPT
    pltpu.sync_copy(z_hbm.at[pl.ds(r0, RPT)], acc_sp.at[pl.ds(r0, RPT)])
    pltpu.sync_copy(t1_hbm.at[0, pl.ds(r0, RPT)], p0_v)
    pltpu.sync_copy(t1_hbm.at[1, pl.ds(r0, RPT)], p1_v)
    pltpu.sync_copy(h1p_hbm.at[pl.ds(r0, RPT)], h1p_v)
    pltpu.sync_copy(dinv_hbm.at[pl.ds(r0, RPT)], dinv_v)
    pltpu.sync_copy(b1_hbm, b1_v)

    @pl.loop(0, RPT, unroll=4)
    def _(r):
        s = p0_v[r, :] + p1_v[r, :] + h1p_v[r, :]
        dv = dinv_v[r, :]
        tr_v[r, :] = jnp.maximum(b1_v[...] + dv * s, 0.0) * dv

    pltpu.sync_copy(tr_v, tab_sp.at[pl.ds(r0, RPT)])

    @pl.when(cid == 0)
    def _():
        pltpu.sync_copy(tr_v, tab_out_hbm.at[pl.ds(r0, RPT)])

    plsc.subcore_barrier()
    _edge_pipeline(t1_hbm.at[0], src_v, dst_v, rows_v, tab_sp, acc_sp,
                   gsems, ssems)
    plsc.subcore_barrier()
    pltpu.sync_copy(acc_sp.at[pl.ds(r0, RPT)], out_hbm.at[cid, pl.ds(r0, RPT)])


_scat2_call = pl.kernel(
    _scat2_body,
    out_type=(jax.ShapeDtypeStruct((NC, NPAD, D_HID), _f32),
              jax.ShapeDtypeStruct((NPAD, D_HID), _f32)),
    mesh=_MESH,
    compiler_params=pltpu.CompilerParams(use_tc_tiling_on_sc=False),
    scratch_types=_SCAT_SCRATCH + [
        pltpu.VMEM((RPT, D_HID), _f32),
        pltpu.VMEM((RPT, D_HID), _f32),
        pltpu.VMEM((RPT, D_HID), _f32),
        pltpu.VMEM((RPT, D_HID), _f32),
        pltpu.VMEM((D_HID,), _f32),
        pltpu.VMEM((RPT, D_HID), _f32),
    ],
)


def _pre_body(x_ref, w_ref, p_ref, o_h1p, o_dinv):
    dinv = lax.rsqrt(1.0 + p_ref[0] + p_ref[1])
    o_dinv[...] = dinv
    h1 = jnp.dot(x_ref[...], w_ref[...], preferred_element_type=_f32)
    o_h1p[pl.ds(0, N), :] = h1 * dinv[:N]
    o_h1p[pl.ds(N, NPAD - N), :] = jnp.zeros((NPAD - N, D_HID), _f32)


def _l2_body(t_ref, a1p_ref, dinv_ref, w2_ref, b2_ref, o_ref):
    u = (t_ref[0] + t_ref[1] + a1p_ref[...])[:N]
    v = jnp.dot(u, w2_ref[...], preferred_element_type=_f32)
    dinv64 = jnp.broadcast_to(dinv_ref[:N, 0:1], (N, D_OUT))
    o = b2_ref[...] + dinv64 * v
    m = jnp.max(o, axis=1, keepdims=True)
    lse = jnp.log(jnp.sum(jnp.exp(o - m), axis=1, keepdims=True)) + m
    o_ref[...] = o - lse


_pre_call = pl.pallas_call(
    _pre_body, out_shape=(jax.ShapeDtypeStruct((NPAD, D_HID), _f32),
                          jax.ShapeDtypeStruct((NPAD, D_HID), _f32)))
_l2_call = pl.pallas_call(
    _l2_body, out_shape=jax.ShapeDtypeStruct((N, D_OUT), _f32))


def kernel(x, edge_index, W1, b1, W2, b2):
    ei3 = edge_index.reshape(2, NGE, GRP)
    z16 = jnp.zeros((NPAD, D_HID), _f32)
    ones_rows = jnp.ones((GRP, D_HID), _f32)

    hist = _hist_call(ei3, ones_rows, z16)
    h1p, dinv = _pre_call(x, W1, hist)
    t1 = _scat1_call(h1p, ei3, z16)
    t2, a1p = _scat2_call(t1, h1p, dinv, b1, ei3, z16)
    out = _l2_call(t2, a1p, dinv, W2, b2.reshape(1, D_OUT))
    return out


# speedup vs baseline: 62.6108x; 1.0208x over previous
import functools

import jax
import jax.numpy as jnp
from jax import lax
from jax.experimental import pallas as pl
from jax.experimental.pallas import tpu as pltpu
from jax.experimental.pallas import tpu_sc as plsc

N = 10000
D_IN = 128
D_HID = 16
D_OUT = 64
E = 320000

NC, NS = 2, 16
NW = NC * NS
GRP = 128
KB = 4
NCH = 20
G = NCH * KB
NGE = E // GRP
NWX = 17
RPT = 640
NPAD = NS * RPT
FL = NPAD // 16
FPT = FL // NS

_MESH = plsc.VectorSubcoreMesh(core_axis_name="c", subcore_axis_name="s")
_f32 = jnp.float32


def _load_idx(ei_hbm, plane, wid, idx_v, pad_val):
    eiv = ei_hbm.at[plane]
    gb = 4 * (19 * wid + jnp.minimum(wid, NWX))
    fill = jnp.full((16,), pad_val, jnp.int32)
    for r in range(G - 4, G):
        for c in range(GRP // 16):
            idx_v[r, pl.ds(c * 16, 16)] = fill
    pltpu.sync_copy(eiv.at[pl.ds(gb, G - 4)], idx_v.at[pl.ds(0, G - 4)])

    @pl.when(wid < NWX)
    def _():
        pltpu.sync_copy(eiv.at[pl.ds(gb + G - 4, 4)], idx_v.at[pl.ds(G - 4, 4)])


EW = E // NW
EWP = NPAD

def _hist_body(dst_hbm, out_hbm, dst_v, priv_v, tmp_v, sum_v, stage_sp):
    cid = lax.axis_index("c")
    sid = lax.axis_index("s")
    wid = sid * NC + cid
    fillN = jnp.full((16,), N, jnp.int32)

    @pl.loop(0, (EWP - EW) // 16, unroll=4)
    def _(r):
        dst_v[pl.ds(EW + r * 16, 16)] = fillN

    pltpu.sync_copy(dst_hbm.at[pl.ds(wid * EW, EW)], dst_v.at[pl.ds(0, EW)])

    @pl.loop(0, NPAD // 16, unroll=8)
    def _(r):
        priv_v[pl.ds(r * 16, 16)] = jnp.zeros((16,), _f32)

    ones16 = jnp.ones((16,), _f32)

    @pl.loop(0, EWP // 16, unroll=4)
    def _(e):
        d = dst_v[pl.ds(e * 16, 16)]
        plsc.addupdate_scatter(priv_v, [d], ones16)

    pltpu.sync_copy(priv_v, stage_sp.at[sid])
    plsc.subcore_barrier()
    nb = sid * RPT
    for t in range(NS):
        pltpu.sync_copy(stage_sp.at[t, pl.ds(nb, RPT)],
                        tmp_v.at[pl.ds(t * RPT, RPT)])

    @pl.loop(0, RPT // 16, unroll=4)
    def _(r):
        s = tmp_v[pl.ds(r * 16, 16)]
        for t in range(1, NS):
            s = s + tmp_v[pl.ds(t * RPT + r * 16, 16)]
        sum_v[pl.ds(r * 16, 16)] = s

    pltpu.sync_copy(sum_v, out_hbm.at[cid, pl.ds(nb, RPT)])


_hist_call = pl.kernel(
    _hist_body,
    out_type=jax.ShapeDtypeStruct((NC, NPAD), _f32),
    mesh=_MESH,
    compiler_params=pltpu.CompilerParams(use_tc_tiling_on_sc=False,
                                         needs_layout_passes=False),
    scratch_types=[
        pltpu.VMEM((EWP,), jnp.int32),
        pltpu.VMEM((NPAD,), _f32),
        pltpu.VMEM((NS * RPT,), _f32),
        pltpu.VMEM((RPT,), _f32),
        pltpu.VMEM_SHARED((NS, NPAD), _f32),
    ],
)


def _edge_pipeline(tab_hbm, src_v, dst_v, rows_v, tab_sp, acc_sp, gsems, ssems):
    def fire_gather(c, p, k):
        pltpu.async_copy(tab_sp.at[src_v.at[c * KB + k]],
                         rows_v.at[p, k], gsems.at[p, k])

    def drain(sem_ref):
        pltpu.make_async_copy(tab_hbm.at[pl.ds(0, GRP)],
                              rows_v.at[0, 0], sem_ref).wait()

    for k in range(KB):
        fire_gather(0, 0, k)

    @pl.loop(0, NCH // 2)
    def _(i):
        for p in range(2):
            c = 2 * i + p
            for k in range(KB):
                drain(gsems.at[p, k])
            @pl.when(c >= 1)
            def _():
                for k in range(KB):
                    drain(ssems.at[1 - p, k])

            @pl.when(c + 1 < NCH)
            def _():
                for k in range(KB):
                    fire_gather(c + 1, 1 - p, k)
            for k in range(KB):
                pltpu.async_copy(rows_v.at[p, k],
                                 acc_sp.at[dst_v.at[c * KB + k]],
                                 ssems.at[p, k], add=True)

    for k in range(KB):
        drain(ssems.at[1, k])


_SCAT_SCRATCH = [
    pltpu.VMEM((G, GRP), jnp.int32),
    pltpu.VMEM((G, GRP), jnp.int32),
    pltpu.VMEM((2, KB, GRP, D_HID), _f32),
    pltpu.VMEM_SHARED((NPAD, D_HID), _f32),
    pltpu.VMEM_SHARED((NPAD, D_HID), _f32),
    pltpu.SemaphoreType.DMA((2, KB)),
    pltpu.SemaphoreType.DMA((2, KB)),
]


def _scat1_body(tab_hbm, ei_hbm, z_hbm, out_hbm,
                src_v, dst_v, rows_v, tab_sp, acc_sp, gsems, ssems):
    cid = lax.axis_index("c")
    sid = lax.axis_index("s")
    wid = sid * NC + cid
    _load_idx(ei_hbm, 0, wid, src_v, 0)
    _load_idx(ei_hbm, 1, wid, dst_v, N)
    r0 = sid * RPT
    pltpu.sync_copy(z_hbm.at[pl.ds(r0, RPT)], acc_sp.at[pl.ds(r0, RPT)])
    pltpu.sync_copy(tab_hbm.at[pl.ds(r0, RPT)], tab_sp.at[pl.ds(r0, RPT)])
    plsc.subcore_barrier()
    _edge_pipeline(tab_hbm, src_v, dst_v, rows_v, tab_sp, acc_sp, gsems, ssems)
    plsc.subcore_barrier()
    pltpu.sync_copy(acc_sp.at[pl.ds(r0, RPT)], out_hbm.at[cid, pl.ds(r0, RPT)])


_scat1_call = pl.kernel(
    _scat1_body,
    out_type=jax.ShapeDtypeStruct((NC, NPAD, D_HID), _f32),
    mesh=_MESH,
    compiler_params=pltpu.CompilerParams(use_tc_tiling_on_sc=False),
    scratch_types=_SCAT_SCRATCH,
)


def _scat2_body(t1_hbm, h1p_hbm, dinv_hbm, b1_hbm, ei_hbm, z_hbm,
                out_hbm, tab_out_hbm,
                src_v, dst_v, rows_v, tab_sp, acc_sp, gsems, ssems,
                p0_v, p1_v, h1p_v, dinv_v, b1_v, tr_v):
    cid = lax.axis_index("c")
    sid = lax.axis_index("s")
    wid = sid * NC + cid
    _load_idx(ei_hbm, 0, wid, src_v, 0)
    _load_idx(ei_hbm, 1, wid, dst_v, N)
    r0 = sid * RPT
    pltpu.sync_copy(z_hbm.at[pl.ds(r0, RPT)], acc_sp.at[pl.ds(r0, RPT)])
    pltpu.sync_copy(t1_hbm.at[0, pl.ds(r0, RPT)], p0_v)
    pltpu.sync_copy(t1_hbm.at[1, pl.ds(r0, RPT)], p1_v)
    pltpu.sync_copy(h1p_hbm.at[pl.ds(r0, RPT)], h1p_v)
    pltpu.sync_copy(dinv_hbm.at[pl.ds(r0, RPT)], dinv_v)
    pltpu.sync_copy(b1_hbm, b1_v)

    @pl.loop(0, RPT, unroll=4)
    def _(r):
        s = p0_v[r, :] + p1_v[r, :] + h1p_v[r, :]
        dv = dinv_v[r, :]
        tr_v[r, :] = jnp.maximum(b1_v[...] + dv * s, 0.0) * dv

    pltpu.sync_copy(tr_v, tab_sp.at[pl.ds(r0, RPT)])

    @pl.when(cid == 0)
    def _():
        pltpu.sync_copy(tr_v, tab_out_hbm.at[pl.ds(r0, RPT)])

    plsc.subcore_barrier()
    _edge_pipeline(t1_hbm.at[0], src_v, dst_v, rows_v, tab_sp, acc_sp,
                   gsems, ssems)
    plsc.subcore_barrier()
    pltpu.sync_copy(acc_sp.at[pl.ds(r0, RPT)], out_hbm.at[cid, pl.ds(r0, RPT)])


_scat2_call = pl.kernel(
    _scat2_body,
    out_type=(jax.ShapeDtypeStruct((NC, NPAD, D_HID), _f32),
              jax.ShapeDtypeStruct((NPAD, D_HID), _f32)),
    mesh=_MESH,
    compiler_params=pltpu.CompilerParams(use_tc_tiling_on_sc=False),
    scratch_types=_SCAT_SCRATCH + [
        pltpu.VMEM((RPT, D_HID), _f32),
        pltpu.VMEM((RPT, D_HID), _f32),
        pltpu.VMEM((RPT, D_HID), _f32),
        pltpu.VMEM((RPT, D_HID), _f32),
        pltpu.VMEM((D_HID,), _f32),
        pltpu.VMEM((RPT, D_HID), _f32),
    ],
)


def _pre_body(x_ref, w_ref, p_ref, o_h1p, o_dinv):
    dinv_f = lax.rsqrt(1.0 + p_ref[0] + p_ref[1])
    dinv = jnp.broadcast_to(dinv_f[:, None], (NPAD, D_HID))
    o_dinv[...] = dinv
    h1 = jnp.dot(x_ref[...], w_ref[...], preferred_element_type=_f32)
    o_h1p[pl.ds(0, N), :] = h1 * dinv[:N]
    o_h1p[pl.ds(N, NPAD - N), :] = jnp.zeros((NPAD - N, D_HID), _f32)


def _l2_body(t_ref, a1p_ref, dinv_ref, w2_ref, b2_ref, o_ref):
    u = (t_ref[0] + t_ref[1] + a1p_ref[...])[:N]
    v = jnp.dot(u, w2_ref[...], preferred_element_type=_f32)
    dinv64 = jnp.broadcast_to(dinv_ref[:N, 0:1], (N, D_OUT))
    o = b2_ref[...] + dinv64 * v
    m = jnp.max(o, axis=1, keepdims=True)
    lse = jnp.log(jnp.sum(jnp.exp(o - m), axis=1, keepdims=True)) + m
    o_ref[...] = o - lse


_pre_call = pl.pallas_call(
    _pre_body, out_shape=(jax.ShapeDtypeStruct((NPAD, D_HID), _f32),
                          jax.ShapeDtypeStruct((NPAD, D_HID), _f32)))
_l2_call = pl.pallas_call(
    _l2_body, out_shape=jax.ShapeDtypeStruct((N, D_OUT), _f32))


def kernel(x, edge_index, W1, b1, W2, b2):
    z16 = jnp.zeros((NPAD, D_HID), _f32)

    ei3 = edge_index.reshape(2, NGE, GRP)
    hist = _hist_call(edge_index[1])
    h1p, dinv = _pre_call(x, W1, hist)
    t1 = _scat1_call(h1p, ei3, z16)
    t2, a1p = _scat2_call(t1, h1p, dinv, b1, ei3, z16)
    out = _l2_call(t2, a1p, dinv, W2, b2.reshape(1, D_OUT))
    return out
